# Initial kernel scaffold; baseline (speedup 1.0000x reference)
import jax
import jax.numpy as jnp
from jax.experimental import pallas as pl


def kernel(logits):
    raise NotImplementedError("write your pallas kernel here")


# speedup vs baseline: 1.8148x; 1.8148x over previous
import functools

import jax
import jax.numpy as jnp
from jax import lax
from jax.experimental import pallas as pl
from jax.experimental.pallas import tpu as pltpu
from jax.experimental.pallas import tpu_sc as plsc

B = 64
V = 100000
K = 50
NS = 4
NB = 512
CAP = 256
L = 16
NVR = V // L
ROWS_PER_W = 2


def _sc_body(logits_hbm, cand_x_hbm, cand_i_hbm, meta_hbm,
             row_v, bins_v, cx_v, ci_v, meta_v):
    wid = lax.axis_index("s") * 2 + lax.axis_index("c")
    lane = lax.iota(jnp.int32, L)
    for r in range(ROWS_PER_W):
        row = wid * ROWS_PER_W + r
        pltpu.sync_copy(logits_hbm.at[row], row_v)

        def p1(i, carry):
            mx, mn = carry
            v = row_v[pl.ds(i * L, L)]
            return jnp.maximum(mx, v), jnp.minimum(mn, v)
        mx, mn = lax.fori_loop(
            0, NVR, p1,
            (jnp.full((L,), -jnp.inf, jnp.float32),
             jnp.full((L,), jnp.inf, jnp.float32)))
        m_hi = jnp.max(mx)
        m_lo = jnp.min(mn)
        scale = jnp.full((L,), jnp.float32(NB)) / jnp.maximum(
            jnp.full((L,), m_hi - m_lo), jnp.float32(1e-30))

        def zbins(i, c):
            bins_v[pl.ds(i * L, L)] = jnp.zeros((L,), jnp.int32)
            return c
        lax.fori_loop(0, NB, zbins, 0)

        ones = jnp.ones((L,), jnp.int32)
        def p2(i, acc):
            v = row_v[pl.ds(i * L, L)]
            b = jnp.minimum(((v - m_lo) * scale).astype(jnp.int32), NB - 1)
            plsc.addupdate_scatter(bins_v, [b * L + lane], ones)
            return acc + jnp.exp(v - m_hi)
        acc = lax.fori_loop(0, NVR, p2, jnp.zeros((L,), jnp.float32))
        s_sum = jnp.sum(acc)

        def t_cond(c):
            j, _, t = c
            return (t < 0) & (j < NB)
        def t_body(c):
            j, cum, t = c
            bb = NB - 1 - j
            cum = cum + jnp.sum(bins_v[pl.ds(bb * L, L)])
            t = jnp.where(cum >= K, bb, t)
            return j + 1, cum, t
        _, _, thr = lax.while_loop(
            t_cond, t_body, (jnp.int32(0), jnp.int32(0), jnp.int32(-1)))
        thr = jnp.maximum(thr, 0)

        def zc(i, c):
            cx_v[pl.ds(i * L, L)] = jnp.zeros((L,), jnp.float32)
            ci_v[pl.ds(i * L, L)] = jnp.zeros((L,), jnp.int32)
            return c
        lax.fori_loop(0, CAP // L, zc, 0)

        def p3(i, off):
            v = row_v[pl.ds(i * L, L)]
            b = jnp.minimum(((v - m_lo) * scale).astype(jnp.int32), NB - 1)
            msk = b >= thr
            offc = jnp.minimum(off, CAP - L)
            plsc.store_compressed(cx_v.at[pl.ds(offc, L)], v, mask=msk)
            plsc.store_compressed(ci_v.at[pl.ds(offc, L)], i * L + lane, mask=msk)
            return off + jnp.sum(msk.astype(jnp.int32))
        off = lax.fori_loop(0, NVR, p3, jnp.int32(0))
        cnt = jnp.minimum(off, CAP)

        meta_v[...] = jnp.where(
            lane == 0, m_hi,
            jnp.where(lane == 1, s_sum,
                      jnp.where(lane == 2, cnt.astype(jnp.float32),
                                jnp.float32(0))))
        pltpu.sync_copy(cx_v, cand_x_hbm.at[row])
        pltpu.sync_copy(ci_v, cand_i_hbm.at[row])
        pltpu.sync_copy(meta_v, meta_hbm.at[row])


_sc_select = functools.partial(
    pl.kernel,
    out_type=(jax.ShapeDtypeStruct((B, CAP), jnp.float32),
              jax.ShapeDtypeStruct((B, CAP), jnp.int32),
              jax.ShapeDtypeStruct((B, L), jnp.float32)),
    mesh=plsc.VectorSubcoreMesh(core_axis_name="c", subcore_axis_name="s"),
    compiler_params=pltpu.CompilerParams(needs_layout_passes=False),
    scratch_types=[
        pltpu.VMEM((V,), jnp.float32),
        pltpu.VMEM((NB * L,), jnp.int32),
        pltpu.VMEM((CAP,), jnp.float32),
        pltpu.VMEM((CAP,), jnp.int32),
        pltpu.VMEM((L,), jnp.float32),
    ],
)(_sc_body)


def _tc_body(cx_ref, ci_ref, meta_ref, g_ref, probs_ref, tok_ref):
    m_hi = meta_ref[:, 0:1]
    s_sum = meta_ref[:, 1:2]
    cnt = meta_ref[:, 2:3].astype(jnp.int32)
    cx = cx_ref[...]
    ci = ci_ref[...]
    col = lax.broadcasted_iota(jnp.int32, (B, CAP), 1)
    e0 = jnp.where(col < cnt, jnp.exp(cx - m_hi), -1.0)
    kcol = lax.broadcasted_iota(jnp.int32, (B, K), 1)
    big = jnp.int32(1 << 30)

    def sel(k, carry):
        e, te, ti = carry
        vm = jnp.max(e, axis=1, keepdims=True)
        pos = jnp.min(jnp.where(e == vm, col, big), axis=1, keepdims=True)
        hit = col == pos
        tok = jnp.sum(jnp.where(hit, ci, 0), axis=1, keepdims=True)
        onek = kcol == k
        te = te + jnp.where(onek, vm, jnp.float32(0))
        ti = ti + jnp.where(onek, tok, 0)
        return jnp.where(hit, -1.0, e), te, ti

    _, te, ti = lax.fori_loop(
        0, K, sel,
        (e0, jnp.zeros((B, K), jnp.float32), jnp.zeros((B, K), jnp.int32)))

    tv = te / s_sum
    fp = tv / jnp.sum(tv)
    probs_ref[...] = fp

    lfp = jnp.log(fp + 1e-20)
    rowi = lax.broadcasted_iota(jnp.int32, (B, K), 0)
    flatid = rowi * K + kcol
    r8 = lax.broadcasted_iota(jnp.int32, (8, 128), 0)
    c128 = lax.broadcasted_iota(jnp.int32, (8, 128), 1)
    tk = jnp.zeros((8, 128), jnp.int32)
    for s in range(NS):
        sc = lfp + g_ref[s]
        mxv = jnp.max(sc)
        f = jnp.min(jnp.where(sc == mxv, flatid, big))
        tok_s = jnp.sum(jnp.where(flatid == f, ti, 0))
        tk = tk + jnp.where((r8 == 0) & (c128 == s), tok_s, 0)
    tok_ref[...] = tk


def kernel(logits):
    cand_x, cand_i, meta = _sc_select(logits)
    g = jax.random.gumbel(
        jax.random.key(42), (NS, B * K), jnp.float32).reshape(NS, B, K)
    fp, tk = pl.pallas_call(
        _tc_body,
        out_shape=(jax.ShapeDtypeStruct((B, K), jnp.float32),
                   jax.ShapeDtypeStruct((8, 128), jnp.int32)),
    )(cand_x, cand_i, meta, g)
    return tk[0, :NS], fp.reshape(-1)


# speedup vs baseline: 2.9157x; 1.6066x over previous
import functools

import jax
import jax.numpy as jnp
from jax import lax
from jax.experimental import pallas as pl
from jax.experimental.pallas import tpu as pltpu
from jax.experimental.pallas import tpu_sc as plsc

B = 64
V = 100000
K = 50
NS = 4
NB = 512
CAP = 256
L = 16
NVR = V // L
ROWS_PER_W = 2


U = 10
NCH = NVR // U


def _sc_body(logits_hbm, cand_x_hbm, cand_i_hbm, meta_hbm,
             row_v, bins_v, cbin_v, cx_v, ci_v, meta_v):
    wid = lax.axis_index("s") * 2 + lax.axis_index("c")
    lane = lax.iota(jnp.int32, L)
    for r in range(ROWS_PER_W):
        row = wid * ROWS_PER_W + r
        pltpu.sync_copy(logits_hbm.at[row], row_v)

        def p1(ci, carry):
            mx, mn = carry
            base = ci * (U * L)
            for u in range(U):
                v = row_v[pl.ds(base + u * L, L)]
                mx = jnp.maximum(mx, v)
                mn = jnp.minimum(mn, v)
            return mx, mn
        mx, mn = lax.fori_loop(
            0, NCH, p1,
            (jnp.full((L,), -jnp.inf, jnp.float32),
             jnp.full((L,), jnp.inf, jnp.float32)))
        m_hi = jnp.max(mx)
        m_lo = jnp.min(mn)
        scale = jnp.full((L,), jnp.float32(NB)) / jnp.maximum(
            jnp.full((L,), m_hi - m_lo), jnp.float32(1e-30))

        def zbins(j, c):
            for u in range(16):
                bins_v[pl.ds((j * 16 + u) * L, L)] = jnp.zeros((L,), jnp.int32)
            return c
        lax.fori_loop(0, NB // 16, zbins, 0)

        ones = jnp.ones((L,), jnp.int32)
        def p2(ci, carry):
            acc_a, acc_b = carry
            base = ci * (U * L)
            bmax = jnp.zeros((L,), jnp.int32)
            for u in range(U):
                v = row_v[pl.ds(base + u * L, L)]
                b = jnp.minimum(((v - m_lo) * scale).astype(jnp.int32), NB - 1)
                bmax = jnp.maximum(bmax, b)
                plsc.addupdate_scatter(bins_v, [b * L + lane], ones)
                e = jnp.exp(v - m_hi)
                if u % 2 == 0:
                    acc_a = acc_a + e
                else:
                    acc_b = acc_b + e
            cbin_v[pl.ds(ci * L, L)] = bmax
            return acc_a, acc_b
        acc_a, acc_b = lax.fori_loop(
            0, NCH, p2,
            (jnp.zeros((L,), jnp.float32), jnp.zeros((L,), jnp.float32)))
        s_sum = jnp.sum(acc_a + acc_b)

        def t_cond(c):
            j, _, t = c
            return (t < 0) & (j < NB)
        def t_body(c):
            j, cum, t = c
            bb = NB - 1 - j
            cum = cum + jnp.sum(bins_v[pl.ds(bb * L, L)])
            t = jnp.where(cum >= K, bb, t)
            return j + 1, cum, t
        _, _, thr = lax.while_loop(
            t_cond, t_body, (jnp.int32(0), jnp.int32(0), jnp.int32(-1)))
        thr = jnp.maximum(thr, 0)

        def zc(i, c):
            cx_v[pl.ds(i * L, L)] = jnp.zeros((L,), jnp.float32)
            ci_v[pl.ds(i * L, L)] = jnp.zeros((L,), jnp.int32)
            return c
        lax.fori_loop(0, CAP // L, zc, 0)

        def p3(ci, off):
            bm = cbin_v[pl.ds(ci * L, L)]
            anyhit = jnp.max(bm) >= thr
            def hit(off):
                base = ci * (U * L)
                for u in range(U):
                    v = row_v[pl.ds(base + u * L, L)]
                    b = jnp.minimum(
                        ((v - m_lo) * scale).astype(jnp.int32), NB - 1)
                    msk = b >= thr
                    offc = jnp.minimum(off, CAP - L)
                    plsc.store_compressed(cx_v.at[pl.ds(offc, L)], v, mask=msk)
                    plsc.store_compressed(
                        ci_v.at[pl.ds(offc, L)], base + u * L + lane, mask=msk)
                    off = off + jnp.sum(msk.astype(jnp.int32))
                return off
            return lax.cond(anyhit, hit, lambda o: o, off)
        off = lax.fori_loop(0, NCH, p3, jnp.int32(0))
        cnt = jnp.minimum(off, CAP)

        meta_v[...] = jnp.where(
            lane == 0, m_hi,
            jnp.where(lane == 1, s_sum,
                      jnp.where(lane == 2, cnt.astype(jnp.float32),
                                jnp.float32(0))))
        pltpu.sync_copy(cx_v, cand_x_hbm.at[row])
        pltpu.sync_copy(ci_v, cand_i_hbm.at[row])
        pltpu.sync_copy(meta_v, meta_hbm.at[row])


_sc_select = functools.partial(
    pl.kernel,
    out_type=(jax.ShapeDtypeStruct((B, CAP), jnp.float32),
              jax.ShapeDtypeStruct((B, CAP), jnp.int32),
              jax.ShapeDtypeStruct((B, L), jnp.float32)),
    mesh=plsc.VectorSubcoreMesh(core_axis_name="c", subcore_axis_name="s"),
    compiler_params=pltpu.CompilerParams(needs_layout_passes=False),
    scratch_types=[
        pltpu.VMEM((V,), jnp.float32),
        pltpu.VMEM((NB * L,), jnp.int32),
        pltpu.VMEM((NCH * L,), jnp.int32),
        pltpu.VMEM((CAP,), jnp.float32),
        pltpu.VMEM((CAP,), jnp.int32),
        pltpu.VMEM((L,), jnp.float32),
    ],
)(_sc_body)


def _tc_body(cx_ref, ci_ref, meta_ref, g_ref, probs_ref, tok_ref):
    m_hi = meta_ref[:, 0:1]
    s_sum = meta_ref[:, 1:2]
    cnt = meta_ref[:, 2:3].astype(jnp.int32)
    cx = cx_ref[...]
    ci = ci_ref[...]
    col = lax.broadcasted_iota(jnp.int32, (B, CAP), 1)
    e0 = jnp.where(col < cnt, jnp.exp(cx - m_hi), -1.0)
    kcol = lax.broadcasted_iota(jnp.int32, (B, K), 1)
    big = jnp.int32(1 << 30)

    def sel(k, carry):
        e, te, ti = carry
        vm = jnp.max(e, axis=1, keepdims=True)
        pos = jnp.min(jnp.where(e == vm, col, big), axis=1, keepdims=True)
        hit = col == pos
        tok = jnp.sum(jnp.where(hit, ci, 0), axis=1, keepdims=True)
        onek = kcol == k
        te = te + jnp.where(onek, vm, jnp.float32(0))
        ti = ti + jnp.where(onek, tok, 0)
        return jnp.where(hit, -1.0, e), te, ti

    _, te, ti = lax.fori_loop(
        0, K, sel,
        (e0, jnp.zeros((B, K), jnp.float32), jnp.zeros((B, K), jnp.int32)))

    tv = te / s_sum
    fp = tv / jnp.sum(tv)
    probs_ref[...] = fp

    lfp = jnp.log(fp + 1e-20)
    rowi = lax.broadcasted_iota(jnp.int32, (B, K), 0)
    flatid = rowi * K + kcol
    r8 = lax.broadcasted_iota(jnp.int32, (8, 128), 0)
    c128 = lax.broadcasted_iota(jnp.int32, (8, 128), 1)
    tk = jnp.zeros((8, 128), jnp.int32)
    for s in range(NS):
        sc = lfp + g_ref[s]
        mxv = jnp.max(sc)
        f = jnp.min(jnp.where(sc == mxv, flatid, big))
        tok_s = jnp.sum(jnp.where(flatid == f, ti, 0))
        tk = tk + jnp.where((r8 == 0) & (c128 == s), tok_s, 0)
    tok_ref[...] = tk


def kernel(logits):
    cand_x, cand_i, meta = _sc_select(logits)
    g = jax.random.gumbel(
        jax.random.key(42), (NS, B * K), jnp.float32).reshape(NS, B, K)
    fp, tk = pl.pallas_call(
        _tc_body,
        out_shape=(jax.ShapeDtypeStruct((B, K), jnp.float32),
                   jax.ShapeDtypeStruct((8, 128), jnp.int32)),
    )(cand_x, cand_i, meta, g)
    return tk[0, :NS], fp.reshape(-1)


# speedup vs baseline: 5.8043x; 1.9907x over previous
import functools

import jax
import jax.numpy as jnp
from jax import lax
from jax.experimental import pallas as pl
from jax.experimental.pallas import tpu as pltpu
from jax.experimental.pallas import tpu_sc as plsc

B = 64
V = 100000
K = 50
NS = 4
NB = 512
CAP = 256
L = 16
NVR = V // L
ROWS_PER_W = 2


U = 10
NCH = NVR // U


def _sc_body(logits_hbm, cand_x_hbm, cand_i_hbm, meta_hbm,
             row_v, bins_v, cbin_v, cx_v, ci_v, meta_v):
    wid = lax.axis_index("s") * 2 + lax.axis_index("c")
    lane = lax.iota(jnp.int32, L)
    lane_base = lane * NB
    for r in range(ROWS_PER_W):
        row = wid * ROWS_PER_W + r
        pltpu.sync_copy(logits_hbm.at[row], row_v)

        def p1(ci, carry):
            mxs = list(carry[:4])
            mns = list(carry[4:])
            base = ci * (U * L)
            vs = [row_v[pl.ds(base + u * L, L)] for u in range(U)]
            for u in range(U):
                mxs[u % 4] = jnp.maximum(mxs[u % 4], vs[u])
                mns[u % 4] = jnp.minimum(mns[u % 4], vs[u])
            return tuple(mxs) + tuple(mns)
        st = lax.fori_loop(
            0, NCH, p1,
            tuple([jnp.full((L,), -jnp.inf, jnp.float32)] * 4
                  + [jnp.full((L,), jnp.inf, jnp.float32)] * 4))
        m_hi = jnp.max(jnp.maximum(jnp.maximum(st[0], st[1]),
                                   jnp.maximum(st[2], st[3])))
        m_lo = jnp.min(jnp.minimum(jnp.minimum(st[4], st[5]),
                                   jnp.minimum(st[6], st[7])))
        scale = jnp.full((L,), jnp.float32(NB)) / jnp.maximum(
            jnp.full((L,), m_hi - m_lo), jnp.float32(1e-30))

        def zbins(j, c):
            for u in range(16):
                bins_v[pl.ds((j * 16 + u) * L, L)] = jnp.zeros((L,), jnp.int32)
            return c
        lax.fori_loop(0, NB // 16, zbins, 0)

        def binify(v):
            return jnp.minimum(((v - m_lo) * scale).astype(jnp.int32), NB - 1)

        ones = jnp.ones((L,), jnp.int32)
        def p2(ci, carry):
            acc_a, acc_b = carry
            base = ci * (U * L)
            vs = [row_v[pl.ds(base + u * L, L)] for u in range(U)]
            bs = [binify(v) for v in vs]
            bmax = jnp.zeros((L,), jnp.int32)
            for u in range(U):
                bmax = jnp.maximum(bmax, bs[u])
                e = jnp.exp(vs[u] - m_hi)
                if u % 2 == 0:
                    acc_a = acc_a + e
                else:
                    acc_b = acc_b + e
            cbin_v[pl.ds(ci * L, L)] = bmax
            for u in range(U):
                plsc.addupdate_scatter(bins_v, [lane_base + bs[u]], ones)
            return acc_a, acc_b
        acc_a, acc_b = lax.fori_loop(
            0, NCH, p2,
            (jnp.zeros((L,), jnp.float32), jnp.zeros((L,), jnp.float32)))
        s_sum = jnp.sum(acc_a + acc_b)

        def t_body(j, carry):
            jj = (NB // 16 - 1) - j
            cum_above, t_best = carry
            tot = bins_v[pl.ds(jj * 16, 16)]
            for l in range(1, L):
                tot = tot + bins_v[pl.ds(l * NB + jj * 16, 16)]
            suf = lax.rev(plsc.cumsum(lax.rev(tot, (0,))), (0,))
            r_cnt = suf + cum_above
            binidx = jj * 16 + lane
            cand = jnp.max(jnp.where(r_cnt >= K, binidx, -1))
            t_best = jnp.maximum(t_best, cand)
            return cum_above + jnp.sum(tot), t_best
        _, thr = lax.fori_loop(
            0, NB // 16, t_body, (jnp.int32(0), jnp.int32(-1)))
        thr = jnp.maximum(thr, 0)

        def zc(i, c):
            cx_v[pl.ds(i * L, L)] = jnp.zeros((L,), jnp.float32)
            ci_v[pl.ds(i * L, L)] = jnp.zeros((L,), jnp.int32)
            return c
        lax.fori_loop(0, CAP // L, zc, 0)

        def p3(ci, off):
            bm = cbin_v[pl.ds(ci * L, L)]
            nhit = plsc.all_reduce_population_count(bm >= thr)
            def hit(off):
                base = ci * (U * L)
                vs = [row_v[pl.ds(base + u * L, L)] for u in range(U)]
                msks = [binify(v) >= thr for v in vs]
                for u in range(U):
                    offc = jnp.minimum(off, CAP - L)
                    plsc.store_compressed(
                        cx_v.at[pl.ds(offc, L)], vs[u], mask=msks[u])
                    plsc.store_compressed(
                        ci_v.at[pl.ds(offc, L)], base + u * L + lane,
                        mask=msks[u])
                    off = off + jnp.sum(msks[u].astype(jnp.int32))
                return off
            return lax.cond(nhit[0] > 0, hit, lambda o: o, off)
        off = lax.fori_loop(0, NCH, p3, jnp.int32(0))
        cnt = jnp.minimum(off, CAP)

        meta_v[...] = jnp.where(
            lane == 0, m_hi,
            jnp.where(lane == 1, s_sum,
                      jnp.where(lane == 2, cnt.astype(jnp.float32),
                                jnp.float32(0))))
        pltpu.sync_copy(cx_v, cand_x_hbm.at[row])
        pltpu.sync_copy(ci_v, cand_i_hbm.at[row])
        pltpu.sync_copy(meta_v, meta_hbm.at[row])


_sc_select = functools.partial(
    pl.kernel,
    out_type=(jax.ShapeDtypeStruct((B, CAP), jnp.float32),
              jax.ShapeDtypeStruct((B, CAP), jnp.int32),
              jax.ShapeDtypeStruct((B, L), jnp.float32)),
    mesh=plsc.VectorSubcoreMesh(core_axis_name="c", subcore_axis_name="s"),
    compiler_params=pltpu.CompilerParams(needs_layout_passes=False),
    scratch_types=[
        pltpu.VMEM((V,), jnp.float32),
        pltpu.VMEM((NB * L,), jnp.int32),
        pltpu.VMEM((NCH * L,), jnp.int32),
        pltpu.VMEM((CAP,), jnp.float32),
        pltpu.VMEM((CAP,), jnp.int32),
        pltpu.VMEM((L,), jnp.float32),
    ],
)(_sc_body)


def _tc_body(cx_ref, ci_ref, meta_ref, g_ref, probs_ref, tok_ref):
    m_hi = meta_ref[:, 0:1]
    s_sum = meta_ref[:, 1:2]
    cnt = meta_ref[:, 2:3].astype(jnp.int32)
    cx = cx_ref[...]
    ci = ci_ref[...]
    col = lax.broadcasted_iota(jnp.int32, (B, CAP), 1)
    e0 = jnp.where(col < cnt, jnp.exp(cx - m_hi), -1.0)
    kcol = lax.broadcasted_iota(jnp.int32, (B, K), 1)
    big = jnp.int32(1 << 30)

    def sel(k, carry):
        e, te, ti = carry
        vm = jnp.max(e, axis=1, keepdims=True)
        pos = jnp.min(jnp.where(e == vm, col, big), axis=1, keepdims=True)
        hit = col == pos
        tok = jnp.sum(jnp.where(hit, ci, 0), axis=1, keepdims=True)
        onek = kcol == k
        te = te + jnp.where(onek, vm, jnp.float32(0))
        ti = ti + jnp.where(onek, tok, 0)
        return jnp.where(hit, -1.0, e), te, ti

    _, te, ti = lax.fori_loop(
        0, K, sel,
        (e0, jnp.zeros((B, K), jnp.float32), jnp.zeros((B, K), jnp.int32)))

    tv = te / s_sum
    fp = tv / jnp.sum(tv)
    probs_ref[...] = fp

    lfp = jnp.log(fp + 1e-20)
    rowi = lax.broadcasted_iota(jnp.int32, (B, K), 0)
    flatid = rowi * K + kcol
    r8 = lax.broadcasted_iota(jnp.int32, (8, 128), 0)
    c128 = lax.broadcasted_iota(jnp.int32, (8, 128), 1)
    tk = jnp.zeros((8, 128), jnp.int32)
    for s in range(NS):
        sc = lfp + g_ref[s]
        mxv = jnp.max(sc)
        f = jnp.min(jnp.where(sc == mxv, flatid, big))
        tok_s = jnp.sum(jnp.where(flatid == f, ti, 0))
        tk = tk + jnp.where((r8 == 0) & (c128 == s), tok_s, 0)
    tok_ref[...] = tk


def kernel(logits):
    cand_x, cand_i, meta = _sc_select(logits)
    g = jax.random.gumbel(
        jax.random.key(42), (NS, B * K), jnp.float32).reshape(NS, B, K)
    fp, tk = pl.pallas_call(
        _tc_body,
        out_shape=(jax.ShapeDtypeStruct((B, K), jnp.float32),
                   jax.ShapeDtypeStruct((8, 128), jnp.int32)),
    )(cand_x, cand_i, meta, g)
    return tk[0, :NS], fp.reshape(-1)
